# baseline (device time: 26735 ns/iter reference)
import jax
import jax.numpy as jnp
from jax import lax
from jax.experimental import pallas as pl
from jax.experimental.pallas import tpu as pltpu

QR = 512
CS = 32
NC = QR // CS
DX = 224
DY = 160
DZ = 128
DXC, DYC, DZC = DX // CS, DY // CS, DZ // CS
ORDER = tuple(range(DXC, NC)) + tuple(range(DXC))
Y_FEED = tuple(range(DXC, DXC + DYC))
Z_FEED = tuple(range(DXC + DYC, NC))
LAG = 4


def kernel(x):
    m, n = x.shape

    def body(x_ref, out_ref, recv_x_ref, recv_y_ref, recv_z_ref,
             sx_send, sx_recv, sy_send, sy_recv, sz_send, sz_recv,
             ready_y, ready_z):
        my_x = lax.axis_index("x")
        my_y = lax.axis_index("y")
        my_z = lax.axis_index("z")
        qy = lax.rem(my_y, 2)
        qz = lax.rem(my_z, 2)
        partner = (1 - my_x, my_y, my_z)
        b_y = (my_x, my_y + 1 - 2 * qy, my_z)
        b_z = (my_x, my_y, my_z + 1 - 2 * qz)

        r_me = (2 * qy + qz) * QR
        r_y = (2 * (1 - qy) + qz) * QR
        r_z = (2 * qy + (1 - qz)) * QR
        r_d = (2 * (1 - qy) + (1 - qz)) * QR

        barrier = pltpu.get_barrier_semaphore()
        pl.semaphore_signal(
            barrier, inc=1, device_id=partner,
            device_id_type=pl.DeviceIdType.MESH,
        )
        pl.semaphore_signal(
            ready_y, inc=1, device_id=b_y,
            device_id_type=pl.DeviceIdType.MESH,
        )
        pl.semaphore_signal(
            ready_z, inc=1, device_id=b_z,
            device_id_type=pl.DeviceIdType.MESH,
        )
        pl.semaphore_wait(barrier, 1)

        def rcopy(src, dst, ssem, rsem, dev):
            return pltpu.make_async_remote_copy(
                src_ref=src, dst_ref=dst, send_sem=ssem, recv_sem=rsem,
                device_id=dev, device_id_type=pl.DeviceIdType.MESH,
            )

        rx = [None] * NC
        for k in ORDER:
            rx[k] = rcopy(x_ref.at[pl.ds(r_me + k * CS, CS), :],
                          recv_x_ref.at[pl.ds(k * CS, CS), :],
                          sx_send.at[k], sx_recv.at[k], partner)
            rx[k].start()
        rxd = rcopy(x_ref.at[pl.ds(r_d, DX), :],
                    recv_x_ref.at[pl.ds(QR, DX), :],
                    sx_send.at[NC], sx_recv.at[NC], partner)
        rxd.start()

        pl.semaphore_wait(ready_y, 1)
        pl.semaphore_wait(ready_z, 1)

        ry, rz = [None] * NC, [None] * NC
        ryd, rzd = [None] * DYC, [None] * DZC

        def fwd(k):
            rx[k].wait_recv()
            ry[k] = rcopy(recv_x_ref.at[pl.ds(k * CS, CS), :],
                          recv_y_ref.at[pl.ds(k * CS, CS), :],
                          sy_send.at[k], sy_recv.at[k], b_y)
            ry[k].start()
            rz[k] = rcopy(recv_x_ref.at[pl.ds(k * CS, CS), :],
                          recv_z_ref.at[pl.ds(k * CS, CS), :],
                          sz_send.at[k], sz_recv.at[k], b_z)
            rz[k].start()

        def relay(c):
            if c in Y_FEED:
                j = c - DXC
                rz[c].wait_recv()
                ryd[j] = rcopy(recv_z_ref.at[pl.ds(c * CS, CS), :],
                               recv_y_ref.at[pl.ds(QR + j * CS, CS), :],
                               sy_send.at[NC + j], sy_recv.at[NC + j], b_y)
                ryd[j].start()
            else:
                j = c - DXC - DYC
                ry[c].wait_recv()
                rzd[j] = rcopy(recv_y_ref.at[pl.ds(c * CS, CS), :],
                               recv_z_ref.at[pl.ds(QR + j * CS, CS), :],
                               sz_send.at[NC + j], sz_recv.at[NC + j], b_z)
                rzd[j].start()

        for i, k in enumerate(ORDER):
            fwd(k)
            if i >= LAG and ORDER[i - LAG] in Y_FEED + Z_FEED:
                relay(ORDER[i - LAG])
        for i in range(len(ORDER) - LAG, len(ORDER)):
            if 0 <= i < len(ORDER) and ORDER[i] in Y_FEED + Z_FEED:
                relay(ORDER[i])

        def add_y(k):
            sl = pl.ds(r_y + k * CS, CS)
            out_ref[sl, :] = x_ref[sl, :] + recv_y_ref[pl.ds(k * CS, CS), :]

        def add_z(k):
            sl = pl.ds(r_z + k * CS, CS)
            out_ref[sl, :] = x_ref[sl, :] + recv_z_ref[pl.ds(k * CS, CS), :]

        for c in Z_FEED:
            add_y(c)
        for c in Y_FEED:
            add_z(c)
        sl = pl.ds(r_me, QR)
        out_ref[sl, :] = x_ref[sl, :] + recv_x_ref[pl.ds(0, QR), :]

        for k in ORDER:
            if k not in Z_FEED:
                ry[k].wait_recv()
                add_y(k)
            if k not in Y_FEED:
                rz[k].wait_recv()
                add_z(k)

        rxd.wait_recv()
        sl = pl.ds(r_d, DX)
        out_ref[sl, :] = x_ref[sl, :] + recv_x_ref[pl.ds(QR, DX), :]

        for j in range(DYC):
            ryd[j].wait_recv()
            sl = pl.ds(r_d + DX + j * CS, CS)
            out_ref[sl, :] = x_ref[sl, :] + recv_y_ref[pl.ds(QR + j * CS, CS), :]
        for j in range(DZC):
            rzd[j].wait_recv()
            sl = pl.ds(r_d + DX + DY + j * CS, CS)
            out_ref[sl, :] = x_ref[sl, :] + recv_z_ref[pl.ds(QR + j * CS, CS), :]

        for r in rx + ry + rz + ryd + rzd + [rxd]:
            r.wait_send()

    return pl.pallas_call(
        body,
        out_shape=jax.ShapeDtypeStruct((m, n), x.dtype),
        in_specs=[pl.BlockSpec(memory_space=pltpu.VMEM)],
        out_specs=pl.BlockSpec(memory_space=pltpu.VMEM),
        scratch_shapes=[
            pltpu.VMEM((QR + DX, n), x.dtype),
            pltpu.VMEM((QR + DY, n), x.dtype),
            pltpu.VMEM((QR + DZ, n), x.dtype),
            pltpu.SemaphoreType.DMA((NC + 1,)),
            pltpu.SemaphoreType.DMA((NC + 1,)),
            pltpu.SemaphoreType.DMA((NC + DYC,)),
            pltpu.SemaphoreType.DMA((NC + DYC,)),
            pltpu.SemaphoreType.DMA((NC + DZC,)),
            pltpu.SemaphoreType.DMA((NC + DZC,)),
            pltpu.SemaphoreType.REGULAR,
            pltpu.SemaphoreType.REGULAR,
        ],
        compiler_params=pltpu.CompilerParams(collective_id=0),
    )(x)


# device time: 26238 ns/iter; 1.0189x vs baseline; 1.0189x over previous
import jax
import jax.numpy as jnp
from jax import lax
from jax.experimental import pallas as pl
from jax.experimental.pallas import tpu as pltpu

QR = 512
CS = 64
NC = QR // CS
DX = 256
DY = 128
DZ = 128
ORDER = (4, 5, 6, 7, 0, 1, 2, 3)


def kernel(x):
    m, n = x.shape

    def body(x_ref, out_ref, recv_x_ref, recv_y_ref, recv_z_ref,
             sx_send, sx_recv, sy_send, sy_recv, sz_send, sz_recv,
             ready_y, ready_z):
        my_x = lax.axis_index("x")
        my_y = lax.axis_index("y")
        my_z = lax.axis_index("z")
        qy = lax.rem(my_y, 2)
        qz = lax.rem(my_z, 2)
        partner = (1 - my_x, my_y, my_z)
        b_y = (my_x, my_y + 1 - 2 * qy, my_z)
        b_z = (my_x, my_y, my_z + 1 - 2 * qz)

        r_me = (2 * qy + qz) * QR
        r_y = (2 * (1 - qy) + qz) * QR
        r_z = (2 * qy + (1 - qz)) * QR
        r_d = (2 * (1 - qy) + (1 - qz)) * QR

        barrier = pltpu.get_barrier_semaphore()
        pl.semaphore_signal(
            barrier, inc=1, device_id=partner,
            device_id_type=pl.DeviceIdType.MESH,
        )
        pl.semaphore_signal(
            ready_y, inc=1, device_id=b_y,
            device_id_type=pl.DeviceIdType.MESH,
        )
        pl.semaphore_signal(
            ready_z, inc=1, device_id=b_z,
            device_id_type=pl.DeviceIdType.MESH,
        )
        pl.semaphore_wait(barrier, 1)

        def rcopy(src, dst, ssem, rsem, dev):
            return pltpu.make_async_remote_copy(
                src_ref=src, dst_ref=dst, send_sem=ssem, recv_sem=rsem,
                device_id=dev, device_id_type=pl.DeviceIdType.MESH,
            )

        rx = [None] * NC
        for k in ORDER:
            rx[k] = rcopy(x_ref.at[pl.ds(r_me + k * CS, CS), :],
                          recv_x_ref.at[pl.ds(k * CS, CS), :],
                          sx_send.at[k], sx_recv.at[k], partner)
            rx[k].start()
        rxd = rcopy(x_ref.at[pl.ds(r_d, DX), :],
                    recv_x_ref.at[pl.ds(QR, DX), :],
                    sx_send.at[NC], sx_recv.at[NC], partner)
        rxd.start()

        ry, rz = [None] * NC, [None] * NC

        def fwd(k):
            rx[k].wait_recv()
            ry[k] = rcopy(recv_x_ref.at[pl.ds(k * CS, CS), :],
                          recv_y_ref.at[pl.ds(k * CS, CS), :],
                          sy_send.at[k], sy_recv.at[k], b_y)
            ry[k].start()
            rz[k] = rcopy(recv_x_ref.at[pl.ds(k * CS, CS), :],
                          recv_z_ref.at[pl.ds(k * CS, CS), :],
                          sz_send.at[k], sz_recv.at[k], b_z)
            rz[k].start()

        pl.semaphore_wait(ready_y, 1)
        pl.semaphore_wait(ready_z, 1)

        fwd(4)
        fwd(5)
        fwd(6)
        rz[4].wait_recv()
        ryd0 = rcopy(recv_z_ref.at[pl.ds(4 * CS, CS), :],
                     recv_y_ref.at[pl.ds(QR, CS), :],
                     sy_send.at[NC], sy_recv.at[NC], b_y)
        ryd0.start()
        fwd(7)
        rz[5].wait_recv()
        ryd1 = rcopy(recv_z_ref.at[pl.ds(5 * CS, CS), :],
                     recv_y_ref.at[pl.ds(QR + CS, CS), :],
                     sy_send.at[NC + 1], sy_recv.at[NC + 1], b_y)
        ryd1.start()
        fwd(0)
        ry[6].wait_recv()
        rzd0 = rcopy(recv_y_ref.at[pl.ds(6 * CS, CS), :],
                     recv_z_ref.at[pl.ds(QR, CS), :],
                     sz_send.at[NC], sz_recv.at[NC], b_z)
        rzd0.start()
        fwd(1)
        ry[7].wait_recv()
        rzd1 = rcopy(recv_y_ref.at[pl.ds(7 * CS, CS), :],
                     recv_z_ref.at[pl.ds(QR + CS, CS), :],
                     sz_send.at[NC + 1], sz_recv.at[NC + 1], b_z)
        rzd1.start()
        fwd(2)
        fwd(3)

        def add_y(k):
            sl = pl.ds(r_y + k * CS, CS)
            out_ref[sl, :] = x_ref[sl, :] + recv_y_ref[pl.ds(k * CS, CS), :]

        def add_z(k):
            sl = pl.ds(r_z + k * CS, CS)
            out_ref[sl, :] = x_ref[sl, :] + recv_z_ref[pl.ds(k * CS, CS), :]

        add_y(6)
        add_y(7)
        add_z(4)
        add_z(5)
        sl = pl.ds(r_me, QR)
        out_ref[sl, :] = x_ref[sl, :] + recv_x_ref[pl.ds(0, QR), :]

        for k in (4, 5):
            ry[k].wait_recv()
            add_y(k)
        for k in (6, 7):
            rz[k].wait_recv()
            add_z(k)
        for k in (0, 1, 2, 3):
            ry[k].wait_recv()
            add_y(k)
            rz[k].wait_recv()
            add_z(k)

        rxd.wait_recv()
        sl = pl.ds(r_d, DX)
        out_ref[sl, :] = x_ref[sl, :] + recv_x_ref[pl.ds(QR, DX), :]

        ryd0.wait_recv()
        sl = pl.ds(r_d + DX, CS)
        out_ref[sl, :] = x_ref[sl, :] + recv_y_ref[pl.ds(QR, CS), :]
        ryd1.wait_recv()
        sl = pl.ds(r_d + DX + CS, CS)
        out_ref[sl, :] = x_ref[sl, :] + recv_y_ref[pl.ds(QR + CS, CS), :]

        rzd0.wait_recv()
        sl = pl.ds(r_d + DX + DY, CS)
        out_ref[sl, :] = x_ref[sl, :] + recv_z_ref[pl.ds(QR, CS), :]
        rzd1.wait_recv()
        sl = pl.ds(r_d + DX + DY + CS, CS)
        out_ref[sl, :] = x_ref[sl, :] + recv_z_ref[pl.ds(QR + CS, CS), :]

        for r in rx + ry + rz + [rxd, ryd0, ryd1, rzd0, rzd1]:
            r.wait_send()

    return pl.pallas_call(
        body,
        out_shape=jax.ShapeDtypeStruct((m, n), x.dtype),
        in_specs=[pl.BlockSpec(memory_space=pltpu.VMEM)],
        out_specs=pl.BlockSpec(memory_space=pltpu.VMEM),
        scratch_shapes=[
            pltpu.VMEM((QR + DX, n), x.dtype),
            pltpu.VMEM((QR + DY, n), x.dtype),
            pltpu.VMEM((QR + DZ, n), x.dtype),
            pltpu.SemaphoreType.DMA((NC + 3,)),
            pltpu.SemaphoreType.DMA((NC + 3,)),
            pltpu.SemaphoreType.DMA((NC + 3,)),
            pltpu.SemaphoreType.DMA((NC + 3,)),
            pltpu.SemaphoreType.DMA((NC + 3,)),
            pltpu.SemaphoreType.DMA((NC + 3,)),
            pltpu.SemaphoreType.REGULAR,
            pltpu.SemaphoreType.REGULAR,
        ],
        compiler_params=pltpu.CompilerParams(collective_id=0),
    )(x)
